# baseline (speedup 1.0000x reference)
import functools

import numpy as np
import jax
import jax.numpy as jnp
from jax import lax
from jax.experimental import pallas as pl
from jax.experimental.pallas import tpu as pltpu

_IRREPS = [(160, 0, 1), (64, 1, -1), (32, 2, 1)]
_NUM_GRAPHS = 512
_EPS = 1e-5

_PRNG = np.random.default_rng(0)
_MEAN_SHIFT = (1.0 + 0.1 * _PRNG.standard_normal(160)).astype(np.float32)
_AFFINE_WEIGHT = (1.0 + 0.1 * _PRNG.standard_normal(256)).astype(np.float32)
_AFFINE_BIAS = (0.1 * _PRNG.standard_normal(160)).astype(np.float32)

_CHUNK = 4096
_SPAD = 256


def _build_constants():
    D = sum(m * (2 * l + 1) for m, l, _ in _IRREPS)
    F = sum(m for m, _, _ in _IRREPS)
    P = np.zeros((D, F), np.float32)
    E = np.zeros((F, D), np.float32)
    dinv = np.zeros((1, F), np.float32)
    shift = np.zeros((1, _SPAD), np.float32)
    bias = np.zeros((1, _SPAD), np.float32)
    col = f = 0
    for mul, l, _ in _IRREPS:
        d = 2 * l + 1
        for _ in range(mul):
            P[col:col + d, f] = 1.0
            E[f, col:col + d] = 1.0
            dinv[0, f] = 1.0 / d
            col += d
            f += 1
    nscal = _IRREPS[0][0]
    assert _IRREPS[0][1] == 0 and _IRREPS[0][2] == 1 and nscal <= _SPAD <= F
    shift[0, :nscal] = _MEAN_SHIFT
    bias[0, :nscal] = _AFFINE_BIAS
    weight = _AFFINE_WEIGHT.reshape(1, F).astype(np.float32)
    return P, E, dinv, shift, weight, bias, D, F


_P, _E, _DINV, _SHIFT, _WEIGHT, _BIAS, _D, _F = _build_constants()
_NSCAL = _IRREPS[0][0]


def _onehot_t(bp, g):
    gids = lax.broadcasted_iota(jnp.int32, (g, 128), 0)
    pieces = [(gids == bp[r:r + 1, :]).astype(jnp.bfloat16)
              for r in range(bp.shape[0])]
    return jnp.concatenate(pieces, axis=1)


def _make_table(acc_ref, dinv_ref, shift_ref, w_ref, e_ref, bias_ref,
                tab_ref, eps):
    a = acc_ref[0] + acc_ref[1]
    sumx = a[:, :_SPAD]
    psq = a[:, _SPAD:2 * _SPAD]
    cnt = a[:, _NSCAL:_NSCAL + 1]
    inv_c = 1.0 / jnp.maximum(cnt, 1.0)
    s = shift_ref[...]
    mean = sumx * inv_c
    corr = (2.0 * s - s * s) * sumx * mean
    norm_f = jnp.maximum((psq - corr) * inv_c, 0.0) * dinv_ref[...]
    scale_f = lax.rsqrt(norm_f + eps) * w_ref[...]
    scale_g = jnp.dot(scale_f, e_ref[...],
                      preferred_element_type=jnp.float32)
    off = bias_ref[...] - (mean * s) * scale_g[:, :_SPAD]
    tab_ref[...] = jnp.concatenate([scale_g, off], axis=1).astype(jnp.bfloat16)


def _stats_kernel(b_ref, x_hbm, p_ref, acc_ref, x_buf, in_sem, *, n_steps):
    c = pl.program_id(0)
    chunk = x_buf.shape[1]
    row0 = c * (n_steps * chunk)
    chunk0 = c * n_steps

    def dma_in(slot, step):
        pltpu.make_async_copy(
            x_hbm.at[pl.ds(row0 + step * chunk, chunk), :],
            x_buf.at[slot], in_sem.at[slot]).start()

    def wait_in(slot):
        pltpu.make_async_copy(
            x_hbm.at[pl.ds(0, chunk), :],
            x_buf.at[slot], in_sem.at[slot]).wait()

    dma_in(0, 0)
    acc_ref[...] = jnp.zeros_like(acc_ref)
    p_b = p_ref[...]
    g = acc_ref.shape[1]

    def body(step, _):
        cur = lax.rem(step, 2)
        nxt = lax.rem(step + 1, 2)

        @pl.when(step + 1 < n_steps)
        def _prefetch():
            dma_in(nxt, step + 1)

        wait_in(cur)
        xb = x_buf[cur].astype(jnp.bfloat16)
        xsqb = xb * xb
        pooled = jnp.dot(xsqb, p_b,
                         preferred_element_type=jnp.float32)
        lane = lax.broadcasted_iota(jnp.int32, (chunk, _SPAD), 1)
        head = jnp.where(lane < _NSCAL, xb[:, :_SPAD], jnp.bfloat16(1.0))
        feats = jnp.concatenate(
            [head, pooled.astype(jnp.bfloat16)], axis=1)
        onehot_t = _onehot_t(b_ref[chunk0 + step], g)
        acc_ref[0] += jnp.dot(onehot_t, feats,
                              preferred_element_type=jnp.float32)
        return ()

    lax.fori_loop(0, n_steps, body, ())


def _apply_kernel(b_ref, x_hbm, acc_ref, dinv_ref, shift_ref, w_ref, e_ref,
                  bias_ref, o_hbm, x_buf, o_buf, tab_ref, in_sem, out_sem,
                  *, n_steps, eps):
    c = pl.program_id(0)
    chunk = x_buf.shape[1]
    row0 = c * (n_steps * chunk)
    chunk0 = c * n_steps

    def dma_in(slot, step):
        pltpu.make_async_copy(
            x_hbm.at[pl.ds(row0 + step * chunk, chunk), :],
            x_buf.at[slot], in_sem.at[slot]).start()

    def wait_in(slot):
        pltpu.make_async_copy(
            x_hbm.at[pl.ds(0, chunk), :],
            x_buf.at[slot], in_sem.at[slot]).wait()

    def dma_out(slot, step):
        pltpu.make_async_copy(
            o_buf.at[slot],
            o_hbm.at[pl.ds(row0 + step * chunk, chunk), :],
            out_sem.at[slot]).start()

    def wait_out(slot):
        pltpu.make_async_copy(
            o_buf.at[slot],
            o_hbm.at[pl.ds(0, chunk), :],
            out_sem.at[slot]).wait()

    dma_in(0, 0)
    _make_table(acc_ref, dinv_ref, shift_ref, w_ref, e_ref, bias_ref,
                tab_ref, eps)
    tab = tab_ref[...]
    g = tab_ref.shape[0]

    def body(step, _):
        cur = lax.rem(step, 2)
        nxt = lax.rem(step + 1, 2)

        @pl.when(step + 1 < n_steps)
        def _prefetch():
            dma_in(nxt, step + 1)

        wait_in(cur)

        @pl.when(step >= 2)
        def _drain():
            wait_out(cur)

        onehot_t = _onehot_t(b_ref[chunk0 + step], g)
        so = lax.dot_general(onehot_t, tab,
                             (((0,), (0,)), ((), ())),
                             preferred_element_type=jnp.float32)
        x = x_buf[cur]
        scale = so[:, :_D]
        off = so[:, _D:]
        lo = x[:, :_SPAD] * scale[:, :_SPAD] + off
        hi = x[:, _SPAD:] * scale[:, _SPAD:]
        o_buf[cur] = jnp.concatenate([lo, hi], axis=1)
        dma_out(cur, step)
        return ()

    lax.fori_loop(0, n_steps, body, ())

    @pl.when(n_steps >= 2)
    def _tail():
        wait_out(lax.rem(n_steps - 2, 2))
    wait_out(lax.rem(n_steps - 1, 2))


def kernel(node_input, batch):
    N, D = node_input.shape
    G = _NUM_GRAPHS
    chunk = _CHUNK
    half = -(-N // (2 * chunk))
    n_pad = 2 * half * chunk

    batch = jnp.asarray(batch, jnp.int32)
    x = node_input
    if n_pad != N:
        batch = jnp.pad(batch, (0, n_pad - N), constant_values=G)
        x = jnp.pad(x, ((0, n_pad - N), (0, 0)))
    bp = batch.reshape(2 * half, chunk // 128, 128)

    p_b = jnp.asarray(_P, jnp.bfloat16)
    e_j = jnp.asarray(_E)
    dinv_j = jnp.asarray(_DINV)
    shift_j = jnp.asarray(_SHIFT)
    w_j = jnp.asarray(_WEIGHT)
    bias_j = jnp.asarray(_BIAS)

    width = 2 * _SPAD
    brows = chunk // 128
    any_spec = pl.BlockSpec(memory_space=pl.ANY)

    acc = pl.pallas_call(
        functools.partial(_stats_kernel, n_steps=half),
        out_shape=jax.ShapeDtypeStruct((2, G, width), jnp.float32),
        grid=(2,),
        in_specs=[
            pl.BlockSpec((2 * half, brows, 128), lambda c: (0, 0, 0)),
            any_spec,
            pl.BlockSpec((_D, _F), lambda c: (0, 0)),
        ],
        out_specs=pl.BlockSpec((1, G, width), lambda c: (c, 0, 0)),
        scratch_shapes=[
            pltpu.VMEM((2, chunk, _D), jnp.float32),
            pltpu.SemaphoreType.DMA((2,)),
        ],
        compiler_params=pltpu.CompilerParams(
            dimension_semantics=("parallel",)),
        cost_estimate=pl.CostEstimate(
            flops=int(2 * n_pad * (G * width + D * _F)),
            transcendentals=0,
            bytes_accessed=int(4 * n_pad * D + 4 * n_pad + 8 * G * width)),
    )(bp, x, p_b)

    out = pl.pallas_call(
        functools.partial(_apply_kernel, n_steps=half, eps=_EPS),
        out_shape=jax.ShapeDtypeStruct((n_pad, D), node_input.dtype),
        grid=(2,),
        in_specs=[
            pl.BlockSpec((2 * half, brows, 128), lambda c: (0, 0, 0)),
            any_spec,
            pl.BlockSpec((2, G, width), lambda c: (0, 0, 0)),
            pl.BlockSpec((1, _F), lambda c: (0, 0)),
            pl.BlockSpec((1, _SPAD), lambda c: (0, 0)),
            pl.BlockSpec((1, _F), lambda c: (0, 0)),
            pl.BlockSpec((_F, _D), lambda c: (0, 0)),
            pl.BlockSpec((1, _SPAD), lambda c: (0, 0)),
        ],
        out_specs=any_spec,
        scratch_shapes=[
            pltpu.VMEM((2, chunk, _D), jnp.float32),
            pltpu.VMEM((2, chunk, _D), jnp.float32),
            pltpu.VMEM((G, _D + _SPAD), jnp.bfloat16),
            pltpu.SemaphoreType.DMA((2,)),
            pltpu.SemaphoreType.DMA((2,)),
        ],
        compiler_params=pltpu.CompilerParams(
            dimension_semantics=("parallel",)),
        cost_estimate=pl.CostEstimate(
            flops=int(2 * n_pad * (G * (D + _SPAD) + D)),
            transcendentals=int(G * _F),
            bytes_accessed=int(8 * n_pad * D + 4 * n_pad + 8 * G * width)),
    )(bp, x, acc, dinv_j, shift_j, w_j, e_j, bias_j)

    return out[:N] if n_pad != N else out


# speedup vs baseline: 1.0594x; 1.0594x over previous
import functools

import numpy as np
import jax
import jax.numpy as jnp
from jax import lax
from jax.experimental import pallas as pl
from jax.experimental.pallas import tpu as pltpu

_IRREPS = [(160, 0, 1), (64, 1, -1), (32, 2, 1)]
_NUM_GRAPHS = 512
_EPS = 1e-5

_PRNG = np.random.default_rng(0)
_MEAN_SHIFT = (1.0 + 0.1 * _PRNG.standard_normal(160)).astype(np.float32)
_AFFINE_WEIGHT = (1.0 + 0.1 * _PRNG.standard_normal(256)).astype(np.float32)
_AFFINE_BIAS = (0.1 * _PRNG.standard_normal(160)).astype(np.float32)

_CHUNK = 4096
_SPAD = 256


def _build_constants():
    D = sum(m * (2 * l + 1) for m, l, _ in _IRREPS)
    F = sum(m for m, _, _ in _IRREPS)
    P = np.zeros((D, F), np.float32)
    E = np.zeros((F, D), np.float32)
    dinv = np.zeros((1, F), np.float32)
    shift = np.zeros((1, _SPAD), np.float32)
    bias = np.zeros((1, _SPAD), np.float32)
    col = f = 0
    for mul, l, _ in _IRREPS:
        d = 2 * l + 1
        for _ in range(mul):
            P[col:col + d, f] = 1.0
            E[f, col:col + d] = 1.0
            dinv[0, f] = 1.0 / d
            col += d
            f += 1
    nscal = _IRREPS[0][0]
    assert _IRREPS[0][1] == 0 and _IRREPS[0][2] == 1 and nscal <= _SPAD <= F
    shift[0, :nscal] = _MEAN_SHIFT
    bias[0, :nscal] = _AFFINE_BIAS
    weight = _AFFINE_WEIGHT.reshape(1, F).astype(np.float32)
    return P, E, dinv, shift, weight, bias, D, F


_P, _E, _DINV, _SHIFT, _WEIGHT, _BIAS, _D, _F = _build_constants()
_NSCAL = _IRREPS[0][0]


def _onehot_t(b_ref, g):
    bp = b_ref[0]
    gids = lax.broadcasted_iota(jnp.int32, (g, 128), 0)
    pieces = [(gids == bp[r:r + 1, :]).astype(jnp.bfloat16)
              for r in range(bp.shape[0])]
    return jnp.concatenate(pieces, axis=1)


def _stats_kernel(b_ref, x_ref, p_ref, acc_ref):
    t = pl.program_id(1)

    @pl.when(t == 0)
    def _init():
        acc_ref[...] = jnp.zeros_like(acc_ref)

    xb = x_ref[...].astype(jnp.bfloat16)
    xsqb = xb * xb
    pooled = jnp.dot(xsqb, p_ref[...],
                     preferred_element_type=jnp.float32)
    lane = lax.broadcasted_iota(jnp.int32, (xb.shape[0], _SPAD), 1)
    head = jnp.where(lane < _NSCAL, xb[:, :_SPAD], jnp.bfloat16(1.0))
    feats = jnp.concatenate(
        [head, pooled.astype(jnp.bfloat16)], axis=1)
    onehot_t = _onehot_t(b_ref, acc_ref.shape[1])
    acc_ref[0] += jnp.dot(onehot_t, feats,
                          preferred_element_type=jnp.float32)


def _make_table(acc_ref, dinv_ref, shift_ref, w_ref, e_ref, bias_ref,
                tab_ref, eps):
    a = acc_ref[0] + acc_ref[1]
    sumx = a[:, :_SPAD]
    psq = a[:, _SPAD:2 * _SPAD]
    cnt = a[:, _NSCAL:_NSCAL + 1]
    inv_c = 1.0 / jnp.maximum(cnt, 1.0)
    s = shift_ref[...]
    mean = sumx * inv_c
    corr = (2.0 * s - s * s) * sumx * mean
    norm_f = jnp.maximum((psq - corr) * inv_c, 0.0) * dinv_ref[...]
    scale_f = lax.rsqrt(norm_f + eps) * w_ref[...]
    scale_g = jnp.dot(scale_f, e_ref[...],
                      preferred_element_type=jnp.float32)
    off = bias_ref[...] - (mean * s) * scale_g[:, :_SPAD]
    tab_ref[...] = jnp.concatenate([scale_g, off], axis=1).astype(jnp.bfloat16)


def _apply_kernel(b_ref, x_ref, acc_ref, dinv_ref, shift_ref, w_ref, e_ref,
                  bias_ref, o_ref, tab_ref, *, eps):
    t = pl.program_id(1)

    @pl.when(t == 0)
    def _finalize():
        _make_table(acc_ref, dinv_ref, shift_ref, w_ref, e_ref, bias_ref,
                    tab_ref, eps)

    onehot_t = _onehot_t(b_ref, tab_ref.shape[0])
    so = lax.dot_general(onehot_t, tab_ref[...],
                         (((0,), (0,)), ((), ())),
                         preferred_element_type=jnp.float32)
    x = x_ref[...]
    d = x.shape[1]
    scale = so[:, :d]
    off = so[:, d:]
    o_ref[:, :_SPAD] = (x[:, :_SPAD] * scale[:, :_SPAD] + off
                        ).astype(o_ref.dtype)
    o_ref[:, _SPAD:] = (x[:, _SPAD:] * scale[:, _SPAD:]).astype(o_ref.dtype)


def kernel(node_input, batch):
    N, D = node_input.shape
    G = _NUM_GRAPHS
    chunk = _CHUNK
    half = -(-N // (2 * chunk))
    n_pad = 2 * half * chunk

    batch = jnp.asarray(batch, jnp.int32)
    x = node_input
    if n_pad != N:
        batch = jnp.pad(batch, (0, n_pad - N), constant_values=G)
        x = jnp.pad(x, ((0, n_pad - N), (0, 0)))
    bp = batch.reshape(2 * half, chunk // 128, 128)

    p_b = jnp.asarray(_P, jnp.bfloat16)
    e_j = jnp.asarray(_E)
    dinv_j = jnp.asarray(_DINV)
    shift_j = jnp.asarray(_SHIFT)
    w_j = jnp.asarray(_WEIGHT)
    bias_j = jnp.asarray(_BIAS)

    width = 2 * _SPAD
    brows = chunk // 128

    acc = pl.pallas_call(
        _stats_kernel,
        out_shape=jax.ShapeDtypeStruct((2, G, width), jnp.float32),
        grid=(2, half),
        in_specs=[
            pl.BlockSpec((1, brows, 128), lambda c, t: (c * half + t, 0, 0)),
            pl.BlockSpec((chunk, D), lambda c, t: (c * half + t, 0)),
            pl.BlockSpec((D, _F), lambda c, t: (0, 0)),
        ],
        out_specs=pl.BlockSpec((1, G, width), lambda c, t: (c, 0, 0)),
        compiler_params=pltpu.CompilerParams(
            dimension_semantics=("parallel", "arbitrary")),
        cost_estimate=pl.CostEstimate(
            flops=int(2 * n_pad * (G * width + D * _F)),
            transcendentals=0,
            bytes_accessed=int(4 * n_pad * D + 4 * n_pad + 8 * G * width)),
    )(bp, x, p_b)

    out = pl.pallas_call(
        functools.partial(_apply_kernel, eps=_EPS),
        out_shape=jax.ShapeDtypeStruct((n_pad, D), node_input.dtype),
        grid=(2, half),
        in_specs=[
            pl.BlockSpec((1, brows, 128), lambda c, t: (c * half + t, 0, 0)),
            pl.BlockSpec((chunk, D), lambda c, t: (c * half + t, 0)),
            pl.BlockSpec((2, G, width), lambda c, t: (0, 0, 0)),
            pl.BlockSpec((1, _F), lambda c, t: (0, 0)),
            pl.BlockSpec((1, _SPAD), lambda c, t: (0, 0)),
            pl.BlockSpec((1, _F), lambda c, t: (0, 0)),
            pl.BlockSpec((_F, _D), lambda c, t: (0, 0)),
            pl.BlockSpec((1, _SPAD), lambda c, t: (0, 0)),
        ],
        out_specs=pl.BlockSpec((chunk, D), lambda c, t: (c * half + t, 0)),
        scratch_shapes=[pltpu.VMEM((G, _D + _SPAD), jnp.bfloat16)],
        compiler_params=pltpu.CompilerParams(
            dimension_semantics=("parallel", "arbitrary")),
        cost_estimate=pl.CostEstimate(
            flops=int(2 * n_pad * (G * (D + _SPAD) + D)),
            transcendentals=int(G * _F),
            bytes_accessed=int(8 * n_pad * D + 4 * n_pad + 8 * G * width)),
    )(bp, x, acc, dinv_j, shift_j, w_j, e_j, bias_j)

    return out[:N] if n_pad != N else out
